# Initial kernel scaffold; baseline (speedup 1.0000x reference)
import jax
import jax.numpy as jnp
from jax.experimental import pallas as pl


def kernel(x, x_lens, enc_Wih0, enc_Whh0, enc_b0, enc_Wih1, enc_Whh1, enc_b1, enc_Wih2, enc_Whh2, enc_b2, pred_emb, pred_Wih0, pred_Whh0, pred_b0, pred_Wih1, pred_Whh1, pred_b1, joint_W1, joint_b1, joint_W2, joint_b2):
    raise NotImplementedError("write your pallas kernel here")


# speedup vs baseline: 2.4430x; 2.4430x over previous
import functools

import jax
import jax.numpy as jnp
from jax.experimental import pallas as pl
from jax.experimental.pallas import tpu as pltpu

T, N, C = 128, 16, 240
ENC_H = 512
PRED_H = 320
JOINT_H = 512
VOCAB = 29
VPAD = 32
BLANK = VOCAB - 1
SOS = VOCAB - 1
MAX_SYMBOLS = 4
MAX_STEPS = (MAX_SYMBOLS + 1) * T
R = MAX_SYMBOLS * T


def _lstm_apply(gates, c, H):
    gi = jax.nn.sigmoid(gates[:, 0 * H:1 * H])
    gf = jax.nn.sigmoid(gates[:, 1 * H:2 * H])
    gg = jnp.tanh(gates[:, 2 * H:3 * H])
    go = jax.nn.sigmoid(gates[:, 3 * H:4 * H])
    c2 = gf * c + gi * gg
    h2 = go * jnp.tanh(c2)
    return h2, c2


def _decoder_kernel(
    one_ref, x_ref, xlens_ref,
    eW0i_ref, eW0h_ref, eb0_ref,
    eW1i_ref, eW1h_ref, eb1_ref,
    eW2i_ref, eW2h_ref, eb2_ref,
    emb_ref,
    pW0i_ref, pW0h_ref, pb0_ref,
    pW1i_ref, pW1h_ref, pb1_ref,
    jW1_ref, jb1_ref,
    jW2_ref, jb2_ref,
    res_ref, ridx_ref,
    f_sc, fi_sc, h_sc, c_sc, t_smem,
):
    s1 = one_ref[0]

    def dot1(a, b):
        return jnp.dot(a, b, preferred_element_type=jnp.float32) * s1

    for l in range(3):
        h_sc[l] = jnp.zeros((N, ENC_H), jnp.float32)
        c_sc[l] = jnp.zeros((N, ENC_H), jnp.float32)

    eWi = (eW0i_ref, eW1i_ref, eW2i_ref)
    eWh = (eW0h_ref, eW1h_ref, eW2h_ref)
    eb = (eb0_ref, eb1_ref, eb2_ref)

    def enc_step(t, _):
        inp = x_ref[t]
        for l in range(3):
            gates = dot1(inp, eWi[l][...]) + dot1(h_sc[l], eWh[l][...]) + eb[l][...]
            h2, c2 = _lstm_apply(gates, c_sc[l], ENC_H)
            h_sc[l] = h2
            c_sc[l] = c2
            inp = h2
        f_sc[t] = inp
        return 0

    jax.lax.fori_loop(0, T, enc_step, 0)

    res_ref[...] = jnp.full((N, R), -1, jnp.int32)
    fi_sc[...] = f_sc[0]
    for n in range(N):
        t_smem[n] = 0

    xlens = xlens_ref[...]
    lane_iota = jax.lax.broadcasted_iota(jnp.int32, (N, VPAD), 1)
    col_iota = jax.lax.broadcasted_iota(jnp.int32, (N, R), 1)

    def dec_step(s, carry):
        time_idx, sym_added, res_idx, pred_g, hg0, hg1, cg0, cg1 = carry
        active = time_idx < xlens

        onehot = (pred_g == lane_iota).astype(jnp.float32)
        emb = jnp.dot(onehot, emb_ref[...],
                      preferred_element_type=jnp.float32, precision="highest")
        g0 = dot1(emb, pW0i_ref[...]) + dot1(hg0, pW0h_ref[...]) + pb0_ref[...]
        h0n, c0n = _lstm_apply(g0, cg0, PRED_H)
        g1 = dot1(h0n, pW1i_ref[...]) + dot1(hg1, pW1h_ref[...]) + pb1_ref[...]
        h1n, c1n = _lstm_apply(g1, cg1, PRED_H)

        cat = jnp.concatenate([fi_sc[...], h1n], axis=-1)
        hidden = jax.nn.relu(dot1(cat, jW1_ref[...]) + jb1_ref[...])
        y = dot1(hidden, jW2_ref[...]) + jb2_ref[...]

        mx = jnp.max(y, axis=1, keepdims=True)
        sym = jnp.min(jnp.where(y == mx, lane_iota, VPAD), axis=1, keepdims=True)

        advance = (sym == BLANK) | (sym_added >= MAX_SYMBOLS)
        emit = active & (~advance)
        adv = active & advance

        new_res_idx = jnp.where(emit, res_idx + 1, res_idx)
        cols = jnp.clip(new_res_idx, 0, R - 1)
        res_ref[...] = jnp.where((col_iota == cols) & emit, sym, res_ref[...])
        pred_g = jnp.where(emit, sym, pred_g)
        hg0 = jnp.where(emit, h0n, hg0)
        hg1 = jnp.where(emit, h1n, hg1)
        cg0 = jnp.where(emit, c0n, cg0)
        cg1 = jnp.where(emit, c1n, cg1)
        sym_added = jnp.where(emit, sym_added + 1, jnp.where(adv, 0, sym_added))
        time_idx = jnp.where(adv, time_idx + 1, time_idx)

        adv_i32 = adv.astype(jnp.int32)
        for n in range(N):
            adv_n = adv_i32[n, 0] != 0

            @pl.when(adv_n)
            def _():
                t_new = t_smem[n] + 1
                t_smem[n] = t_new
                tc = jnp.minimum(t_new, T - 1)
                fi_sc[pl.ds(n, 1), :] = f_sc[tc, pl.ds(n, 1), :]

        return (time_idx, sym_added, new_res_idx, pred_g, hg0, hg1, cg0, cg1)

    zi = jnp.zeros((N, 1), jnp.int32)
    zh = jnp.zeros((N, PRED_H), jnp.float32)
    carry = (zi, zi, jnp.full((N, 1), -1, jnp.int32),
             jnp.full((N, 1), SOS, jnp.int32), zh, zh, zh, zh)
    carry = jax.lax.fori_loop(0, MAX_STEPS, dec_step, carry)
    ridx_ref[...] = carry[2] + 1


@functools.partial(jax.jit, static_argnames=("interpret",))
def kernel(x, x_lens,
           enc_Wih0, enc_Whh0, enc_b0,
           enc_Wih1, enc_Whh1, enc_b1,
           enc_Wih2, enc_Whh2, enc_b2,
           pred_emb,
           pred_Wih0, pred_Whh0, pred_b0,
           pred_Wih1, pred_Whh1, pred_b1,
           joint_W1, joint_b1, joint_W2, joint_b2,
           interpret=False):
    f32 = jnp.float32
    emb_p = jnp.zeros((VPAD, PRED_H), f32).at[:VOCAB].set(pred_emb)
    jW1T = joint_W1.T
    jW2T = jnp.full((JOINT_H, VPAD), 0.0, f32).at[:, :VOCAB].set(joint_W2.T)
    jb2p = jnp.full((1, VPAD), -1e30, f32).at[0, :VOCAB].set(joint_b2)

    args = (
        jnp.ones((1,), f32),
        x, x_lens.reshape(N, 1).astype(jnp.int32),
        enc_Wih0.T, enc_Whh0.T, enc_b0.reshape(1, -1),
        enc_Wih1.T, enc_Whh1.T, enc_b1.reshape(1, -1),
        enc_Wih2.T, enc_Whh2.T, enc_b2.reshape(1, -1),
        emb_p,
        pred_Wih0.T, pred_Whh0.T, pred_b0.reshape(1, -1),
        pred_Wih1.T, pred_Whh1.T, pred_b1.reshape(1, -1),
        jW1T, joint_b1.reshape(1, -1),
        jW2T, jb2p,
    )

    res, ridx = pl.pallas_call(
        _decoder_kernel,
        in_specs=[pl.BlockSpec(memory_space=pltpu.SMEM)]
        + [pl.BlockSpec(memory_space=pltpu.VMEM)] * (len(args) - 1),
        out_shape=[
            jax.ShapeDtypeStruct((N, R), jnp.int32),
            jax.ShapeDtypeStruct((N, 1), jnp.int32),
        ],
        scratch_shapes=[
            pltpu.VMEM((T, N, ENC_H), f32),
            pltpu.VMEM((N, ENC_H), f32),
            pltpu.VMEM((3, N, ENC_H), f32),
            pltpu.VMEM((3, N, ENC_H), f32),
            pltpu.SMEM((N,), jnp.int32),
        ],
        interpret=interpret,
    )(*args)
    return res, ridx.reshape(N)
